# baseline (speedup 1.0000x reference)
import functools

import jax
import jax.numpy as jnp
from jax import lax
from jax.experimental import pallas as pl
from jax.experimental.pallas import tpu as pltpu
from jax.experimental.pallas import tpu_sc as plsc

N_NODES = 10000
N_EDGES = 320000
D_FEAT = 128
D_EDGE = 16
D_HID = 128

NH = 2
EH = N_EDGES // NH

EPT = EH // 16
GCHUNK = 80
GNCHUNK = EPT // GCHUNK
TROWS = 624

NW = 32
EPW = EH // NW
SCHUNK = 40
SNCHUNK = EPW // SCHUNK
ACC_ROWS = 10240
RPS = ACC_ROWS // 16


def _tables_body(nodes_ref, wr_ref, ws_ref, be1_ref, p_ref):
    nb = nodes_ref[...]
    p_ref[0] = (
        jnp.dot(nb, wr_ref[...], preferred_element_type=jnp.float32)
        + be1_ref[...]
    )
    p_ref[1] = jnp.dot(nb, ws_ref[...], preferred_element_type=jnp.float32)


def _edge_body(e_ref, gr_ref, gs_ref, we_ref, we2_ref, be2_ref, ae_ref,
               out_ref):
    pre = (
        jnp.dot(e_ref[...], we_ref[...], preferred_element_type=jnp.float32)
        + gr_ref[0]
        + gs_ref[0]
    )
    h = jnp.maximum(pre, 0.0)
    out = jnp.dot(h, we2_ref[...], preferred_element_type=jnp.float32)
    out_ref[...] = (out + be2_ref[...]) * ae_ref[...]


def _node_body(p00_ref, p01_ref, p10_ref, p11_ref, nodes_ref, wa_ref,
               wn_ref, bn1_ref, wn2_ref, bn2_ref, an_ref, out_ref):
    aggr = (p00_ref[...] + p01_ref[...]) + (p10_ref[...] + p11_ref[...])
    nb = nodes_ref[...]
    hn = jnp.maximum(
        jnp.dot(aggr, wa_ref[...], preferred_element_type=jnp.float32)
        + jnp.dot(nb, wn_ref[...], preferred_element_type=jnp.float32)
        + bn1_ref[...],
        0.0,
    )
    dn = jnp.dot(hn, wn2_ref[...], preferred_element_type=jnp.float32)
    out_ref[...] = nb + (dn + bn2_ref[...]) * an_ref[...]


def _gather_body(tab_hbm, idx_hbm, g_hbm,
                 spm, idx_v, buf_a, buf_b, gsem_a, gsem_b, wsem_a, wsem_b):
    cid = lax.axis_index("c")
    sid = lax.axis_index("s")
    base = sid * EPT
    pltpu.sync_copy(tab_hbm.at[cid, pl.ds(sid * TROWS, TROWS)],
                    spm.at[pl.ds(sid * TROWS, TROWS)])

    @pl.when(sid == 15)
    def _stage_tail():
        pltpu.sync_copy(tab_hbm.at[cid, pl.ds(16 * TROWS, 16)],
                        spm.at[pl.ds(16 * TROWS, 16)])

    pltpu.sync_copy(idx_hbm.at[cid, sid], idx_v)
    plsc.subcore_barrier()
    plsc.subcore_barrier()

    def fire_gather(c, buf, sem):
        pltpu.async_copy(spm.at[idx_v.at[pl.ds(c * GCHUNK, GCHUNK)]],
                         buf, sem)

    def wait_gather(buf, sem):
        pltpu.make_async_copy(spm.at[pl.ds(0, GCHUNK)], buf, sem).wait()

    def fire_write(c, buf, sem):
        pltpu.async_copy(buf, g_hbm.at[cid, pl.ds(base + c * GCHUNK, GCHUNK)],
                         sem)

    def wait_write(buf, sem):
        pltpu.make_async_copy(buf, g_hbm.at[cid, pl.ds(0, GCHUNK)],
                              sem).wait()

    fire_gather(0, buf_a, gsem_a)
    fire_gather(1, buf_b, gsem_b)

    def body(t, c):
        wait_gather(buf_a, gsem_a)
        fire_write(2 * t, buf_a, wsem_a)
        wait_gather(buf_b, gsem_b)
        fire_write(2 * t + 1, buf_b, wsem_b)
        wait_write(buf_a, wsem_a)
        fire_gather(2 * t + 2, buf_a, gsem_a)
        wait_write(buf_b, wsem_b)
        fire_gather(2 * t + 3, buf_b, gsem_b)
        return c

    lax.fori_loop(0, (GNCHUNK - 3) // 2, body, 0)
    wait_gather(buf_a, gsem_a)
    fire_write(GNCHUNK - 3, buf_a, wsem_a)
    wait_gather(buf_b, gsem_b)
    fire_write(GNCHUNK - 2, buf_b, wsem_b)
    wait_write(buf_a, wsem_a)
    fire_gather(GNCHUNK - 1, buf_a, gsem_a)
    wait_gather(buf_a, gsem_a)
    fire_write(GNCHUNK - 1, buf_a, wsem_a)
    wait_write(buf_a, wsem_a)
    wait_write(buf_b, wsem_b)


def _scatter_body(ne_hbm, ridx_hbm, zeros_hbm, part_hbm,
                  idx_v, rows_a, rows_b, acc, lsem_a, lsem_b):
    cid = lax.axis_index("c")
    sid = lax.axis_index("s")
    wid = sid * 2 + cid
    base = wid * EPW
    pltpu.sync_copy(ridx_hbm.at[wid], idx_v)
    pltpu.sync_copy(zeros_hbm, acc.at[pl.ds(sid * RPS, RPS)])
    plsc.subcore_barrier()

    def fire_load(c, buf, sem):
        pltpu.async_copy(ne_hbm.at[pl.ds(base + c * SCHUNK, SCHUNK)], buf,
                         sem)

    def wait_load(buf, sem):
        pltpu.make_async_copy(ne_hbm.at[pl.ds(0, SCHUNK)], buf, sem).wait()

    def scat(c, buf):
        pltpu.sync_copy(buf, acc.at[idx_v.at[c]], add=True)

    fire_load(0, rows_a, lsem_a)
    fire_load(1, rows_b, lsem_b)

    def body(t, c):
        wait_load(rows_a, lsem_a)
        scat(2 * t, rows_a)
        fire_load(2 * t + 2, rows_a, lsem_a)
        wait_load(rows_b, lsem_b)
        scat(2 * t + 1, rows_b)
        fire_load(2 * t + 3, rows_b, lsem_b)
        return c

    lax.fori_loop(0, (SNCHUNK - 3) // 2, body, 0)
    wait_load(rows_a, lsem_a)
    scat(SNCHUNK - 3, rows_a)
    fire_load(SNCHUNK - 1, rows_a, lsem_a)
    wait_load(rows_b, lsem_b)
    scat(SNCHUNK - 2, rows_b)
    wait_load(rows_a, lsem_a)
    scat(SNCHUNK - 1, rows_a)
    plsc.subcore_barrier()
    pltpu.sync_copy(acc.at[pl.ds(sid * RPS, RPS)],
                    part_hbm.at[cid, pl.ds(sid * RPS, RPS)])


def _full(i):
    return (0, 0)


def kernel(nodes, edges, receivers, senders, active_nodes, active_edges,
           W_e1, b_e1, W_e2, b_e2, W_n1, b_n1, W_n2, b_n2):
    recv = receivers.astype(jnp.int32)
    send = senders.astype(jnp.int32)
    W_E = W_e1[:D_EDGE]
    W_R = W_e1[D_EDGE:D_EDGE + D_FEAT]
    W_S = W_e1[D_EDGE + D_FEAT:]
    W_A = W_n1[:D_HID]
    W_N = W_n1[D_HID:]
    be1 = b_e1.reshape(1, D_HID)
    be2 = b_e2.reshape(1, D_HID)
    bn1 = b_n1.reshape(1, D_HID)
    bn2 = b_n2.reshape(1, D_FEAT)
    ae = active_edges.reshape(N_EDGES, 1)
    an = active_nodes.reshape(N_NODES, 1)

    f32 = jnp.float32
    NB = 2000
    EB = 2000

    p_tab = pl.pallas_call(
        _tables_body,
        grid=(N_NODES // NB,),
        in_specs=[
            pl.BlockSpec((NB, D_FEAT), lambda i: (i, 0)),
            pl.BlockSpec((D_FEAT, D_HID), _full),
            pl.BlockSpec((D_FEAT, D_HID), _full),
            pl.BlockSpec((1, D_HID), _full),
        ],
        out_specs=pl.BlockSpec((2, NB, D_HID), lambda i: (0, i, 0)),
        out_shape=jax.ShapeDtypeStruct((2, N_NODES, D_HID), f32),
    )(nodes, W_R, W_S, be1)

    mesh = plsc.VectorSubcoreMesh(core_axis_name="c", subcore_axis_name="s")
    gather = pl.kernel(
        _gather_body,
        out_type=jax.ShapeDtypeStruct((2, EH, D_HID), f32),
        mesh=mesh,
        scratch_types=[
            pltpu.VMEM_SHARED((N_NODES, D_HID), f32),
            pltpu.VMEM((EPT,), jnp.int32),
            pltpu.VMEM((GCHUNK, D_HID), f32),
            pltpu.VMEM((GCHUNK, D_HID), f32),
            pltpu.SemaphoreType.DMA,
            pltpu.SemaphoreType.DMA,
            pltpu.SemaphoreType.DMA,
            pltpu.SemaphoreType.DMA,
        ],
    )
    scatter = pl.kernel(
        _scatter_body,
        out_type=jax.ShapeDtypeStruct((2, ACC_ROWS, D_HID), f32),
        mesh=mesh,
        scratch_types=[
            pltpu.VMEM((SNCHUNK, SCHUNK), jnp.int32),
            pltpu.VMEM((SCHUNK, D_HID), f32),
            pltpu.VMEM((SCHUNK, D_HID), f32),
            pltpu.VMEM_SHARED((ACC_ROWS, D_HID), f32),
            pltpu.SemaphoreType.DMA,
            pltpu.SemaphoreType.DMA,
        ],
    )
    zeros = jnp.zeros((RPS, D_HID), f32)

    parts = []
    for h in range(NH):
        lo = h * EH
        recv_h = lax.dynamic_slice_in_dim(recv, lo, EH)
        send_h = lax.dynamic_slice_in_dim(send, lo, EH)
        idx2 = jnp.stack([recv_h.reshape(16, EPT), send_h.reshape(16, EPT)])
        g = gather(p_tab, idx2)

        ne_h = pl.pallas_call(
            _edge_body,
            grid=(EH // EB,),
            in_specs=[
                pl.BlockSpec((EB, D_EDGE), lambda i: (i, 0)),
                pl.BlockSpec((1, EB, D_HID), lambda i: (0, i, 0)),
                pl.BlockSpec((1, EB, D_HID), lambda i: (1, i, 0)),
                pl.BlockSpec((D_EDGE, D_HID), _full),
                pl.BlockSpec((D_HID, D_HID), _full),
                pl.BlockSpec((1, D_HID), _full),
                pl.BlockSpec((EB, 1), lambda i: (i, 0)),
            ],
            out_specs=pl.BlockSpec((EB, D_HID), lambda i: (i, 0)),
            out_shape=jax.ShapeDtypeStruct((EH, D_HID), f32),
        )(lax.dynamic_slice_in_dim(edges, lo, EH), g, g, W_E, W_e2, be2,
          lax.dynamic_slice_in_dim(ae, lo, EH))

        part = scatter(ne_h, recv_h.reshape(NW, SNCHUNK, SCHUNK), zeros)
        parts.append(part)
        if h == 0:
            ne0 = ne_h
        else:
            ne1 = ne_h

    new_edges = jnp.concatenate([ne0, ne1], axis=0)

    new_nodes = pl.pallas_call(
        _node_body,
        grid=(N_NODES // NB,),
        in_specs=[
            pl.BlockSpec((NB, D_HID), lambda i: (i, 0)),
            pl.BlockSpec((NB, D_HID), lambda i: (i, 0)),
            pl.BlockSpec((NB, D_HID), lambda i: (i, 0)),
            pl.BlockSpec((NB, D_HID), lambda i: (i, 0)),
            pl.BlockSpec((NB, D_FEAT), lambda i: (i, 0)),
            pl.BlockSpec((D_HID, D_HID), _full),
            pl.BlockSpec((D_FEAT, D_HID), _full),
            pl.BlockSpec((1, D_HID), _full),
            pl.BlockSpec((D_HID, D_FEAT), _full),
            pl.BlockSpec((1, D_FEAT), _full),
            pl.BlockSpec((NB, 1), lambda i: (i, 0)),
        ],
        out_specs=pl.BlockSpec((NB, D_FEAT), lambda i: (i, 0)),
        out_shape=jax.ShapeDtypeStruct((N_NODES, D_FEAT), f32),
    )(parts[0][0, :N_NODES], parts[0][1, :N_NODES],
      parts[1][0, :N_NODES], parts[1][1, :N_NODES],
      nodes, W_A, W_N, bn1, W_n2, bn2, an)

    return (new_nodes, new_edges)


# speedup vs baseline: 1.3192x; 1.3192x over previous
import functools

import jax
import jax.numpy as jnp
from jax import lax
from jax.experimental import pallas as pl
from jax.experimental.pallas import tpu as pltpu
from jax.experimental.pallas import tpu_sc as plsc

N_NODES = 10000
N_EDGES = 320000
D_FEAT = 128
D_EDGE = 16
D_HID = 128

EPT = N_EDGES // 16
GCHUNK = 80
GNCHUNK = EPT // GCHUNK
TROWS = 624

NW = 32
EPW = N_EDGES // NW
SCHUNK = 80
SNCHUNK = EPW // SCHUNK
ACC_ROWS = 10240
RPS = ACC_ROWS // 16


def _tables_body(nodes_ref, wr_ref, ws_ref, be1_ref, p_ref):
    nb = nodes_ref[...]
    p_ref[0] = (
        jnp.dot(nb, wr_ref[...], preferred_element_type=jnp.float32)
        + be1_ref[...]
    )
    p_ref[1] = jnp.dot(nb, ws_ref[...], preferred_element_type=jnp.float32)


def _edge_body(e_ref, gr_ref, gs_ref, we_ref, we2_ref, be2_ref, ae_ref,
               out_ref):
    pre = (
        jnp.dot(e_ref[...], we_ref[...], preferred_element_type=jnp.float32)
        + gr_ref[0]
        + gs_ref[0]
    )
    h = jnp.maximum(pre, 0.0)
    out = jnp.dot(h, we2_ref[...], preferred_element_type=jnp.float32)
    out_ref[...] = (out + be2_ref[...]) * ae_ref[...]


def _node_body(p0_ref, p1_ref, nodes_ref, wa_ref,
               wn_ref, bn1_ref, wn2_ref, bn2_ref, an_ref, out_ref):
    aggr = p0_ref[...] + p1_ref[...]
    nb = nodes_ref[...]
    hn = jnp.maximum(
        jnp.dot(aggr, wa_ref[...], preferred_element_type=jnp.float32)
        + jnp.dot(nb, wn_ref[...], preferred_element_type=jnp.float32)
        + bn1_ref[...],
        0.0,
    )
    dn = jnp.dot(hn, wn2_ref[...], preferred_element_type=jnp.float32)
    out_ref[...] = nb + (dn + bn2_ref[...]) * an_ref[...]


def _gather_body(tab_hbm, idx_hbm, g_hbm,
                 spm, idx_v, buf_a, buf_b, gsem_a, gsem_b, wsem_a, wsem_b):
    cid = lax.axis_index("c")
    sid = lax.axis_index("s")
    base = sid * EPT
    pltpu.sync_copy(tab_hbm.at[cid, pl.ds(sid * TROWS, TROWS)],
                    spm.at[pl.ds(sid * TROWS, TROWS)])

    @pl.when(sid == 15)
    def _stage_tail():
        pltpu.sync_copy(tab_hbm.at[cid, pl.ds(16 * TROWS, 16)],
                        spm.at[pl.ds(16 * TROWS, 16)])

    pltpu.sync_copy(idx_hbm.at[cid, sid], idx_v)
    plsc.subcore_barrier()
    plsc.subcore_barrier()

    def fire_gather(c, buf, sem):
        pltpu.async_copy(spm.at[idx_v.at[pl.ds(c * GCHUNK, GCHUNK)]],
                         buf, sem)

    def wait_gather(buf, sem):
        pltpu.make_async_copy(spm.at[pl.ds(0, GCHUNK)], buf, sem).wait()

    def fire_write(c, buf, sem):
        pltpu.async_copy(buf, g_hbm.at[cid, pl.ds(base + c * GCHUNK, GCHUNK)],
                         sem)

    def wait_write(buf, sem):
        pltpu.make_async_copy(buf, g_hbm.at[cid, pl.ds(0, GCHUNK)],
                              sem).wait()

    fire_gather(0, buf_a, gsem_a)
    fire_gather(1, buf_b, gsem_b)

    def body(t, c):
        wait_gather(buf_a, gsem_a)
        fire_write(2 * t, buf_a, wsem_a)
        wait_gather(buf_b, gsem_b)
        fire_write(2 * t + 1, buf_b, wsem_b)
        wait_write(buf_a, wsem_a)
        fire_gather(2 * t + 2, buf_a, gsem_a)
        wait_write(buf_b, wsem_b)
        fire_gather(2 * t + 3, buf_b, gsem_b)
        return c

    lax.fori_loop(0, (GNCHUNK - 2) // 2, body, 0)
    wait_gather(buf_a, gsem_a)
    fire_write(GNCHUNK - 2, buf_a, wsem_a)
    wait_gather(buf_b, gsem_b)
    fire_write(GNCHUNK - 1, buf_b, wsem_b)
    wait_write(buf_a, wsem_a)
    wait_write(buf_b, wsem_b)


def _scatter_body(ne_hbm, ridx_hbm, zeros_hbm, part_hbm,
                  idx_v, rows_a, rows_b, acc, lsem_a, lsem_b):
    cid = lax.axis_index("c")
    sid = lax.axis_index("s")
    wid = sid * 2 + cid
    base = wid * EPW
    pltpu.sync_copy(ridx_hbm.at[wid], idx_v)
    pltpu.sync_copy(zeros_hbm, acc.at[pl.ds(sid * RPS, RPS)])
    plsc.subcore_barrier()

    def fire_load(c, buf, sem):
        pltpu.async_copy(ne_hbm.at[pl.ds(base + c * SCHUNK, SCHUNK)], buf,
                         sem)

    def wait_load(buf, sem):
        pltpu.make_async_copy(ne_hbm.at[pl.ds(0, SCHUNK)], buf, sem).wait()

    def scat(c, buf):
        pltpu.sync_copy(buf, acc.at[idx_v.at[c]], add=True)

    fire_load(0, rows_a, lsem_a)
    fire_load(1, rows_b, lsem_b)

    def body(t, c):
        wait_load(rows_a, lsem_a)
        scat(2 * t, rows_a)
        fire_load(2 * t + 2, rows_a, lsem_a)
        wait_load(rows_b, lsem_b)
        scat(2 * t + 1, rows_b)
        fire_load(2 * t + 3, rows_b, lsem_b)
        return c

    lax.fori_loop(0, (SNCHUNK - 3) // 2, body, 0)
    wait_load(rows_a, lsem_a)
    scat(SNCHUNK - 3, rows_a)
    fire_load(SNCHUNK - 1, rows_a, lsem_a)
    wait_load(rows_b, lsem_b)
    scat(SNCHUNK - 2, rows_b)
    wait_load(rows_a, lsem_a)
    scat(SNCHUNK - 1, rows_a)
    plsc.subcore_barrier()
    pltpu.sync_copy(acc.at[pl.ds(sid * RPS, RPS)],
                    part_hbm.at[cid, pl.ds(sid * RPS, RPS)])


def _full(i):
    return (0, 0)


def kernel(nodes, edges, receivers, senders, active_nodes, active_edges,
           W_e1, b_e1, W_e2, b_e2, W_n1, b_n1, W_n2, b_n2):
    recv = receivers.astype(jnp.int32)
    send = senders.astype(jnp.int32)
    W_E = W_e1[:D_EDGE]
    W_R = W_e1[D_EDGE:D_EDGE + D_FEAT]
    W_S = W_e1[D_EDGE + D_FEAT:]
    W_A = W_n1[:D_HID]
    W_N = W_n1[D_HID:]
    be1 = b_e1.reshape(1, D_HID)
    be2 = b_e2.reshape(1, D_HID)
    bn1 = b_n1.reshape(1, D_HID)
    bn2 = b_n2.reshape(1, D_FEAT)
    ae = active_edges.reshape(N_EDGES, 1)
    an = active_nodes.reshape(N_NODES, 1)

    f32 = jnp.float32
    NB = 2000
    EB = 4000

    p_tab = pl.pallas_call(
        _tables_body,
        grid=(N_NODES // NB,),
        in_specs=[
            pl.BlockSpec((NB, D_FEAT), lambda i: (i, 0)),
            pl.BlockSpec((D_FEAT, D_HID), _full),
            pl.BlockSpec((D_FEAT, D_HID), _full),
            pl.BlockSpec((1, D_HID), _full),
        ],
        out_specs=pl.BlockSpec((2, NB, D_HID), lambda i: (0, i, 0)),
        out_shape=jax.ShapeDtypeStruct((2, N_NODES, D_HID), f32),
    )(nodes, W_R, W_S, be1)

    mesh = plsc.VectorSubcoreMesh(core_axis_name="c", subcore_axis_name="s")
    idx2 = jnp.stack([recv.reshape(16, EPT), send.reshape(16, EPT)])
    gather = pl.kernel(
        _gather_body,
        out_type=jax.ShapeDtypeStruct((2, N_EDGES, D_HID), f32),
        mesh=mesh,
        scratch_types=[
            pltpu.VMEM_SHARED((N_NODES, D_HID), f32),
            pltpu.VMEM((EPT,), jnp.int32),
            pltpu.VMEM((GCHUNK, D_HID), f32),
            pltpu.VMEM((GCHUNK, D_HID), f32),
            pltpu.SemaphoreType.DMA,
            pltpu.SemaphoreType.DMA,
            pltpu.SemaphoreType.DMA,
            pltpu.SemaphoreType.DMA,
        ],
    )
    g = gather(p_tab, idx2)

    new_edges = pl.pallas_call(
        _edge_body,
        grid=(N_EDGES // EB,),
        in_specs=[
            pl.BlockSpec((EB, D_EDGE), lambda i: (i, 0)),
            pl.BlockSpec((1, EB, D_HID), lambda i: (0, i, 0)),
            pl.BlockSpec((1, EB, D_HID), lambda i: (1, i, 0)),
            pl.BlockSpec((D_EDGE, D_HID), _full),
            pl.BlockSpec((D_HID, D_HID), _full),
            pl.BlockSpec((1, D_HID), _full),
            pl.BlockSpec((EB, 1), lambda i: (i, 0)),
        ],
        out_specs=pl.BlockSpec((EB, D_HID), lambda i: (i, 0)),
        out_shape=jax.ShapeDtypeStruct((N_EDGES, D_HID), f32),
    )(edges, g, g, W_E, W_e2, be2, ae)

    scatter = pl.kernel(
        _scatter_body,
        out_type=jax.ShapeDtypeStruct((2, ACC_ROWS, D_HID), f32),
        mesh=mesh,
        scratch_types=[
            pltpu.VMEM((SNCHUNK, SCHUNK), jnp.int32),
            pltpu.VMEM((SCHUNK, D_HID), f32),
            pltpu.VMEM((SCHUNK, D_HID), f32),
            pltpu.VMEM_SHARED((ACC_ROWS, D_HID), f32),
            pltpu.SemaphoreType.DMA,
            pltpu.SemaphoreType.DMA,
        ],
    )
    part = scatter(new_edges, recv.reshape(NW, SNCHUNK, SCHUNK),
                   jnp.zeros((RPS, D_HID), f32))

    new_nodes = pl.pallas_call(
        _node_body,
        grid=(N_NODES // NB,),
        in_specs=[
            pl.BlockSpec((NB, D_HID), lambda i: (i, 0)),
            pl.BlockSpec((NB, D_HID), lambda i: (i, 0)),
            pl.BlockSpec((NB, D_FEAT), lambda i: (i, 0)),
            pl.BlockSpec((D_HID, D_HID), _full),
            pl.BlockSpec((D_FEAT, D_HID), _full),
            pl.BlockSpec((1, D_HID), _full),
            pl.BlockSpec((D_HID, D_FEAT), _full),
            pl.BlockSpec((1, D_FEAT), _full),
            pl.BlockSpec((NB, 1), lambda i: (i, 0)),
        ],
        out_specs=pl.BlockSpec((NB, D_FEAT), lambda i: (i, 0)),
        out_shape=jax.ShapeDtypeStruct((N_NODES, D_FEAT), f32),
    )(part[0, :N_NODES], part[1, :N_NODES], nodes, W_A, W_N, bn1, W_n2,
      bn2, an)

    return (new_nodes, new_edges)
